# Initial kernel scaffold; baseline (speedup 1.0000x reference)
import jax
import jax.numpy as jnp
from jax.experimental import pallas as pl


def kernel(token_ids, pos_onehot, W):
    raise NotImplementedError("write your pallas kernel here")


# speedup vs baseline: 2.4901x; 2.4901x over previous
import functools

import jax
import jax.numpy as jnp
from jax import lax
from jax.experimental import pallas as pl
from jax.experimental.pallas import tpu as pltpu
from jax.experimental.pallas import tpu_sc as plsc

_D = 128
_P = 32
_NC = 2
_NS = 16
_NW = _NC * _NS
_CHUNK = 128


def _emb_combine(idx, pos, tab, *, n_rows):
    rows_per_w = n_rows // _NW
    n_chunks = rows_per_w // _CHUNK
    mesh = plsc.VectorSubcoreMesh(core_axis_name="c", subcore_axis_name="s")

    @functools.partial(
        pl.kernel,
        out_type=jax.ShapeDtypeStruct((n_rows, _D + _P), jnp.float32),
        mesh=mesh,
        scratch_types=[
            pltpu.VMEM((_CHUNK,), jnp.int32),
            pltpu.VMEM((_CHUNK, _D), jnp.float32),
            pltpu.VMEM((_CHUNK, _P), jnp.float32),
            pltpu.SemaphoreType.DMA,
        ],
    )
    def body(idx_hbm, pos_hbm, tab_hbm, out_hbm, idx_v, word_v, pos_v, sem):
        wid = lax.axis_index("s") * _NC + lax.axis_index("c")
        base0 = wid * rows_per_w

        def step(i, carry):
            base = base0 + i * _CHUNK
            pltpu.sync_copy(idx_hbm.at[pl.ds(base, _CHUNK)], idx_v)
            gat = pltpu.async_copy(tab_hbm.at[idx_v], word_v, sem)
            pltpu.sync_copy(pos_hbm.at[pl.ds(base, _CHUNK)], pos_v)
            gat.wait()
            pltpu.sync_copy(word_v, out_hbm.at[pl.ds(base, _CHUNK), pl.ds(0, _D)])
            pltpu.sync_copy(pos_v, out_hbm.at[pl.ds(base, _CHUNK), pl.ds(_D, _P)])
            return carry

        lax.fori_loop(0, n_chunks, step, 0)

    return body(idx, pos, tab)


def kernel(token_ids, pos_onehot, W):
    b, s = token_ids.shape
    n = b * s
    idx = token_ids.reshape(n).astype(jnp.int32)
    pos = pos_onehot.reshape(n, _P).astype(jnp.float32)
    out = _emb_combine(idx, pos, W, n_rows=n)
    return out.reshape(b, s, _D + _P)


# speedup vs baseline: 2.8802x; 1.1567x over previous
import functools

import jax
import jax.numpy as jnp
from jax import lax
from jax.experimental import pallas as pl
from jax.experimental.pallas import tpu as pltpu
from jax.experimental.pallas import tpu_sc as plsc

_D = 128
_P = 32
_NC = 2
_NS = 16
_NW = _NC * _NS
_C = 128


def _emb_combine(idx, pos, tab, *, n_rows):
    rows_per_w = n_rows // _NW
    n_chunks = rows_per_w // _C
    assert n_chunks % 2 == 0 and n_chunks >= 4
    mesh = plsc.VectorSubcoreMesh(core_axis_name="c", subcore_axis_name="s")

    @functools.partial(
        pl.kernel,
        out_type=jax.ShapeDtypeStruct((n_rows, _D + _P), jnp.float32),
        mesh=mesh,
        scratch_types=[
            pltpu.VMEM((n_chunks, _C), jnp.int32),
            pltpu.VMEM((_C, _D), jnp.float32),
            pltpu.VMEM((_C, _D), jnp.float32),
            pltpu.VMEM((_C, _P), jnp.float32),
            pltpu.VMEM((_C, _P), jnp.float32),
            pltpu.SemaphoreType.DMA,
            pltpu.SemaphoreType.DMA,
            pltpu.SemaphoreType.DMA,
            pltpu.SemaphoreType.DMA,
            pltpu.SemaphoreType.DMA,
            pltpu.SemaphoreType.DMA,
        ],
    )
    def body(idx_hbm, pos_hbm, tab_hbm, out_hbm, idx_all,
             word_v0, word_v1, pos_v0, pos_v1,
             sg0, sg1, sp0, sp1, sw0, sw1):
        word_v = (word_v0, word_v1)
        pos_v = (pos_v0, pos_v1)
        sg = (sg0, sg1)
        sp = (sp0, sp1)
        sw = (sw0, sw1)

        wid = lax.axis_index("s") * _NC + lax.axis_index("c")
        base0 = wid * rows_per_w

        def start_inputs(i, slot):
            base = base0 + i * _C
            pltpu.async_copy(tab_hbm.at[idx_all.at[i]], word_v[slot], sg[slot])
            pltpu.async_copy(pos_hbm.at[pl.ds(base, _C)], pos_v[slot], sp[slot])

        def wait_inputs(slot):
            pltpu.make_async_copy(
                tab_hbm.at[idx_all.at[0]], word_v[slot], sg[slot]).wait()
            pltpu.make_async_copy(
                pos_hbm.at[pl.ds(0, _C)], pos_v[slot], sp[slot]).wait()

        def start_writes(i, slot):
            base = base0 + i * _C
            pltpu.async_copy(
                word_v[slot], out_hbm.at[pl.ds(base, _C), pl.ds(0, _D)], sw[slot])
            pltpu.async_copy(
                pos_v[slot], out_hbm.at[pl.ds(base, _C), pl.ds(_D, _P)], sw[slot])

        def wait_writes(slot):
            pltpu.make_async_copy(
                word_v[slot], out_hbm.at[pl.ds(0, _C), pl.ds(0, _D)], sw[slot]).wait()
            pltpu.make_async_copy(
                pos_v[slot], out_hbm.at[pl.ds(0, _C), pl.ds(_D, _P)], sw[slot]).wait()

        def step(i, slot, first=False, last=False):
            if not first:
                wait_writes(1 - slot)
            if not last:
                start_inputs(i + 1, 1 - slot)
            wait_inputs(slot)
            start_writes(i, slot)

        pltpu.sync_copy(idx_hbm.at[wid], idx_all)

        start_inputs(0, 0)
        step(0, 0, first=True)
        step(1, 1)

        def pair(j, carry):
            step(2 * j, 0)
            step(2 * j + 1, 1)
            return carry

        lax.fori_loop(1, n_chunks // 2 - 1, pair, 0)

        step(n_chunks - 2, 0)
        step(n_chunks - 1, 1, last=True)
        wait_writes(1)

    idx3 = idx.reshape(_NW, n_chunks, _C)
    return body(idx3, pos, tab)


def kernel(token_ids, pos_onehot, W):
    b, s = token_ids.shape
    n = b * s
    idx = token_ids.reshape(n).astype(jnp.int32)
    pos = pos_onehot.reshape(n, _P).astype(jnp.float32)
    out = _emb_combine(idx, pos, W, n_rows=n)
    return out.reshape(b, s, _D + _P)
